# Initial kernel scaffold; baseline (speedup 1.0000x reference)
import jax
import jax.numpy as jnp
from jax.experimental import pallas as pl


def kernel(x, edge_index, edge_weight, W, b):
    raise NotImplementedError("write your pallas kernel here")


# speedup vs baseline: 4.7899x; 4.7899x over previous
import functools

import jax
import jax.numpy as jnp
from jax import lax
from jax.experimental import pallas as pl
from jax.experimental.pallas import tpu as pltpu
from jax.experimental.pallas import tpu_sc as plsc

N_NODES = 10000
N_PAD = 10240
D_IN = 256
DQ = 64
E_EDGES = 160000
E_ALL = E_EDGES + N_NODES
NTILES = 16
KB = 128
NCHUNK = 84
EPT = NCHUNK * KB
E_PAD = EPT * NTILES
ROWS_PT = N_PAD // NTILES
BM = 1024


def _mm_body(x_ref, w_ref, y0_ref, y1_ref, y2_ref, y3_ref):
    acc = lax.dot_general(x_ref[...], w_ref[...], (((1,), (1,)), ((), ())),
                          preferred_element_type=jnp.float32)
    y0_ref[...] = acc[:, 0 * DQ:1 * DQ]
    y1_ref[...] = acc[:, 1 * DQ:2 * DQ]
    y2_ref[...] = acc[:, 2 * DQ:3 * DQ]
    y3_ref[...] = acc[:, 3 * DQ:4 * DQ]


_SPLAT_DNUMS = lax.GatherDimensionNumbers(
    offset_dims=(), collapsed_slice_dims=(0,), start_index_map=(0,))


def _splat(vec16, l):
    return lax.gather(vec16, jnp.full((16, 1), l, jnp.int32), _SPLAT_DNUMS,
                      (1,), mode=lax.GatherScatterMode.PROMISE_IN_BOUNDS)


def _fast_rsqrt(d):
    i = plsc.bitcast(d, jnp.int32)
    i = jnp.int32(0x5F3759DF) - (i >> 1)
    r = plsc.bitcast(i, jnp.float32)
    for _ in range(3):
        r = r * (1.5 - 0.5 * d * r * r)
    return r


def _sc_body(ni_hbm, no_hbm, ew_hbm, y0_hbm, y1_hbm, y2_hbm, y3_hbm, b_hbm,
             u0_hbm, u1_hbm, u2_hbm, u3_hbm,
             ni_v, no_v, ew_v, norm_v, rdi_v, rdo_v, rows_v, bias_v, zvec_v,
             degi_sh, dego_sh, acc_sh, sem):
    c = lax.axis_index("c")
    s = lax.axis_index("s")
    zeros16 = jnp.zeros((16,), jnp.float32)

    pltpu.sync_copy(ni_hbm.at[s], ni_v)
    pltpu.sync_copy(no_hbm.at[s], no_v)
    pltpu.sync_copy(ew_hbm.at[s], ew_v)

    def _zvec(i, carry):
        zvec_v[pl.ds(i * 16, 16)] = zeros16
        return carry
    lax.fori_loop(0, ROWS_PT // 16, _zvec, 0)
    pltpu.sync_copy(zvec_v, degi_sh.at[pl.ds(s * ROWS_PT, ROWS_PT)])
    pltpu.sync_copy(zvec_v, dego_sh.at[pl.ds(s * ROWS_PT, ROWS_PT)])

    plsc.subcore_barrier()

    def _deg(j, carry):
        pltpu.sync_copy(ew_v.at[j], degi_sh.at[ni_v.at[j]], add=True)
        pltpu.sync_copy(ew_v.at[j], dego_sh.at[no_v.at[j]], add=True)
        return carry
    lax.fori_loop(0, NCHUNK, _deg, 0)

    plsc.subcore_barrier()

    pltpu.sync_copy(degi_sh, rdi_v)
    pltpu.sync_copy(dego_sh, rdo_v)

    def _rs(i, carry):
        sl = pl.ds(i * 16, 16)
        rdi_v[sl] = _fast_rsqrt(rdi_v[sl])
        rdo_v[sl] = _fast_rsqrt(rdo_v[sl])
        return carry
    lax.fori_loop(0, N_PAD // 16, _rs, 0)

    def _nrm(j, carry):
        for g in range(KB // 16):
            sl = pl.ds(g * 16, 16)
            a = plsc.load_gather(rdi_v, [ni_v[j, sl]])
            bb = plsc.load_gather(rdo_v, [no_v[j, sl]])
            norm_v[j, sl] = (ew_v[j, sl] * a) * bb
        return carry
    lax.fori_loop(0, NCHUNK, _nrm, 0)

    def _pass(y_hbm, u_hbm, bi):
        def _zrows(i, carry):
            rows_v[0, i >> 2, pl.ds((i & 3) * 16, 16)] = zeros16
            return carry
        lax.fori_loop(0, KB * DQ // 16, _zrows, 0)
        for k in range(ROWS_PT // KB):
            pltpu.sync_copy(rows_v.at[0],
                            acc_sh.at[pl.ds(s * ROWS_PT + k * KB, KB)])
        pltpu.sync_copy(b_hbm.at[bi], bias_v)
        plsc.subcore_barrier()

        def _chunk(j, carry):
            pltpu.async_copy(y_hbm.at[ni_v.at[j]], rows_v.at[0], sem).wait()

            def _scale(eg, carry2):
                nv = norm_v[j, pl.ds(eg * 16, 16)]
                for l in range(16):
                    e = eg * 16 + l
                    spl = _splat(nv, l)
                    for f in range(DQ // 16):
                        fs = pl.ds(f * 16, 16)
                        rows_v[0, e, fs] = rows_v[0, e, fs] * spl
                return carry2
            lax.fori_loop(0, KB // 16, _scale, 0)

            pltpu.sync_copy(rows_v.at[0], acc_sh.at[no_v.at[j]], add=True)
            return carry
        lax.fori_loop(0, NCHUNK, _chunk, 0)

        plsc.subcore_barrier()

        for k in range(ROWS_PT // KB):
            off = s * ROWS_PT + k * KB
            pltpu.sync_copy(acc_sh.at[pl.ds(off, KB)], rows_v.at[1])

            def _br(r, carry3):
                for f in range(DQ // 16):
                    fs = pl.ds(f * 16, 16)
                    v = rows_v[1, r, fs] + bias_v[fs]
                    rows_v[1, r, fs] = jnp.maximum(v, 0.0)
                return carry3
            lax.fori_loop(0, KB, _br, 0)
            pltpu.sync_copy(rows_v.at[1], u_hbm.at[pl.ds(off, KB)])

    @pl.when(c == 0)
    def _():
        _pass(y0_hbm, u0_hbm, 0)
        _pass(y1_hbm, u1_hbm, 1)

    @pl.when(c == 1)
    def _():
        _pass(y2_hbm, u2_hbm, 2)
        _pass(y3_hbm, u3_hbm, 3)


@functools.lru_cache(maxsize=1)
def _sc_agg():
    return pl.kernel(
        _sc_body,
        out_type=(jax.ShapeDtypeStruct((N_PAD, DQ), jnp.float32),) * 4,
        mesh=plsc.VectorSubcoreMesh(core_axis_name="c", subcore_axis_name="s",
                                    num_cores=2, num_subcores=NTILES),
        scratch_types=[
            pltpu.VMEM((NCHUNK, KB), jnp.int32),
            pltpu.VMEM((NCHUNK, KB), jnp.int32),
            pltpu.VMEM((NCHUNK, KB), jnp.float32),
            pltpu.VMEM((NCHUNK, KB), jnp.float32),
            pltpu.VMEM((N_PAD,), jnp.float32),
            pltpu.VMEM((N_PAD,), jnp.float32),
            pltpu.VMEM((2, KB, DQ), jnp.float32),
            pltpu.VMEM((DQ,), jnp.float32),
            pltpu.VMEM((ROWS_PT,), jnp.float32),
            pltpu.VMEM_SHARED((N_PAD,), jnp.float32),
            pltpu.VMEM_SHARED((N_PAD,), jnp.float32),
            pltpu.VMEM_SHARED((N_PAD, DQ), jnp.float32),
            pltpu.SemaphoreType.DMA,
        ],
        compiler_params=pltpu.CompilerParams(needs_layout_passes=False,
                                             use_tc_tiling_on_sc=False),
    )


def kernel(x, edge_index, edge_weight, W, b):
    ni = edge_index[0]
    no = edge_index[1]
    loop = jnp.arange(N_NODES, dtype=jnp.int32)
    pad_e = E_PAD - E_ALL
    dummy = jnp.full((pad_e,), N_NODES, jnp.int32)
    ni_all = jnp.concatenate([ni, loop, dummy]).reshape(NTILES, NCHUNK, KB)
    no_all = jnp.concatenate([no, loop, dummy]).reshape(NTILES, NCHUNK, KB)
    ew_all = jnp.concatenate(
        [edge_weight, jnp.ones((N_NODES,), jnp.float32),
         jnp.zeros((pad_e,), jnp.float32)]).reshape(NTILES, NCHUNK, KB)
    x_pad = jnp.pad(x, ((0, N_PAD - N_NODES), (0, 0)))

    ys = pl.pallas_call(
        _mm_body,
        grid=(N_PAD // BM,),
        in_specs=[pl.BlockSpec((BM, D_IN), lambda i: (i, 0)),
                  pl.BlockSpec((D_IN, D_IN), lambda i: (0, 0))],
        out_specs=[pl.BlockSpec((BM, DQ), lambda i: (i, 0))] * 4,
        out_shape=[jax.ShapeDtypeStruct((N_PAD, DQ), jnp.float32)] * 4,
    )(x_pad, W)

    us = _sc_agg()(ni_all, no_all, ew_all, *ys, b.reshape(4, DQ))
    return jnp.concatenate([u[:N_NODES] for u in us], axis=1)
